# baseline (speedup 1.0000x reference)
import functools

import jax
import jax.numpy as jnp
from jax import lax
from jax.experimental import pallas as pl
from jax.experimental.pallas import tpu as pltpu
from jax.experimental.pallas import tpu_sc as plsc

BATCH = 16384
EMBED_DIM = 64
NUM_WORKERS = 32
NUM_CORES = 2
SC_ROWS = 8192
TC_ROWS = BATCH - SC_ROWS
BPW = SC_ROWS // NUM_WORKERS


def _sc_body(user_hbm, pos_hbm, neg_hbm, uemb_hbm, iemb_hbm,
             pos_out, neg_out,
             uq_v, iq_v, jq_v,
             u_rows, i_rows, j_rows,
             pos_v, neg_v, semu, semi, semj):
    wid = lax.axis_index("s") * NUM_CORES + lax.axis_index("c")
    base = wid * BPW

    pltpu.sync_copy(user_hbm.at[pl.ds(base, BPW)], uq_v)
    pltpu.sync_copy(pos_hbm.at[pl.ds(base, BPW)], iq_v)
    pltpu.sync_copy(neg_hbm.at[pl.ds(base, BPW)], jq_v)

    lanes = lax.iota(jnp.int32, 16)

    def fire(g, carry):
        b0 = g * 16
        ru = uq_v[pl.ds(b0, 16)]
        ri = iq_v[pl.ds(b0, 16)]
        rj = jq_v[pl.ds(b0, 16)]
        for b in range(16):
            pltpu.async_copy(uemb_hbm.at[pl.ds(ru[b], 1)],
                             u_rows.at[pl.ds(b0 + b, 1)], semu)
            pltpu.async_copy(iemb_hbm.at[pl.ds(ri[b], 1)],
                             i_rows.at[pl.ds(b0 + b, 1)], semi)
            pltpu.async_copy(iemb_hbm.at[pl.ds(rj[b], 1)],
                             j_rows.at[pl.ds(b0 + b, 1)], semj)
        return carry

    lax.fori_loop(0, BPW // 16, fire, 0)

    pltpu.make_async_copy(uemb_hbm.at[pl.ds(0, BPW)], u_rows, semu).wait()
    pltpu.make_async_copy(uemb_hbm.at[pl.ds(0, BPW)], i_rows, semi).wait()
    pltpu.make_async_copy(uemb_hbm.at[pl.ds(0, BPW)], j_rows, semj).wait()

    def group(g, carry):
        b0 = g * 16
        p_acc = jnp.zeros((16,), jnp.float32)
        n_acc = jnp.zeros((16,), jnp.float32)
        for b in range(16):
            u0 = u_rows[b0 + b, pl.ds(0, 16)]
            u1 = u_rows[b0 + b, pl.ds(16, 16)]
            u2 = u_rows[b0 + b, pl.ds(32, 16)]
            u3 = u_rows[b0 + b, pl.ds(48, 16)]
            i0 = i_rows[b0 + b, pl.ds(0, 16)]
            i1 = i_rows[b0 + b, pl.ds(16, 16)]
            i2 = i_rows[b0 + b, pl.ds(32, 16)]
            i3 = i_rows[b0 + b, pl.ds(48, 16)]
            j0 = j_rows[b0 + b, pl.ds(0, 16)]
            j1 = j_rows[b0 + b, pl.ds(16, 16)]
            j2 = j_rows[b0 + b, pl.ds(32, 16)]
            j3 = j_rows[b0 + b, pl.ds(48, 16)]
            p = (u0 * i0 + u1 * i1) + (u2 * i2 + u3 * i3)
            n = (u0 * j0 + u1 * j1) + (u2 * j2 + u3 * j3)
            sel = lanes == b
            p_acc = jnp.where(sel, jnp.sum(p), p_acc)
            n_acc = jnp.where(sel, jnp.sum(n), n_acc)
        pos_v[pl.ds(b0, 16)] = p_acc
        neg_v[pl.ds(b0, 16)] = n_acc
        return carry

    lax.fori_loop(0, BPW // 16, group, 0)

    pltpu.sync_copy(pos_v, pos_out.at[pl.ds(base, BPW)])
    pltpu.sync_copy(neg_v, neg_out.at[pl.ds(base, BPW)])


def _tc_body(uidx, iidx, jidx, uemb, iemb, pos_o, neg_o,
             ubuf, ibuf, jbuf, sem):
    def issue(k, carry):
        pltpu.make_async_copy(uemb.at[pl.ds(uidx[k], 1)],
                              ubuf.at[pl.ds(k, 1)], sem).start()
        pltpu.make_async_copy(iemb.at[pl.ds(iidx[k], 1)],
                              ibuf.at[pl.ds(k, 1)], sem).start()
        pltpu.make_async_copy(iemb.at[pl.ds(jidx[k], 1)],
                              jbuf.at[pl.ds(k, 1)], sem).start()
        return carry

    lax.fori_loop(0, TC_ROWS, issue, 0, unroll=4)

    pltpu.make_async_copy(uemb.at[pl.ds(0, TC_ROWS)], ubuf, sem).wait()
    pltpu.make_async_copy(uemb.at[pl.ds(0, TC_ROWS)], ibuf, sem).wait()
    pltpu.make_async_copy(uemb.at[pl.ds(0, TC_ROWS)], jbuf, sem).wait()

    u = ubuf[...]
    iv = ibuf[...]
    jv = jbuf[...]
    pos_o[...] = jnp.sum(u * iv, axis=1)
    neg_o[...] = jnp.sum(u * jv, axis=1)


@jax.jit
def kernel(user, pos_item, neg_item, user_emb, item_emb):
    mesh = plsc.VectorSubcoreMesh(core_axis_name="c", subcore_axis_name="s")
    sc_f = pl.kernel(
        _sc_body,
        mesh=mesh,
        compiler_params=pltpu.CompilerParams(needs_layout_passes=False),
        out_type=(
            jax.ShapeDtypeStruct((SC_ROWS,), jnp.float32),
            jax.ShapeDtypeStruct((SC_ROWS,), jnp.float32),
        ),
        scratch_types=[
            pltpu.VMEM((BPW,), jnp.int32),
            pltpu.VMEM((BPW,), jnp.int32),
            pltpu.VMEM((BPW,), jnp.int32),
            pltpu.VMEM((BPW, EMBED_DIM), jnp.float32),
            pltpu.VMEM((BPW, EMBED_DIM), jnp.float32),
            pltpu.VMEM((BPW, EMBED_DIM), jnp.float32),
            pltpu.VMEM((BPW,), jnp.float32),
            pltpu.VMEM((BPW,), jnp.float32),
            pltpu.SemaphoreType.DMA,
            pltpu.SemaphoreType.DMA,
            pltpu.SemaphoreType.DMA,
        ],
    )
    pos_sc, neg_sc = sc_f(user[:SC_ROWS], pos_item[:SC_ROWS],
                          neg_item[:SC_ROWS], user_emb, item_emb)

    tc_f = pl.pallas_call(
        _tc_body,
        grid_spec=pltpu.PrefetchScalarGridSpec(
            num_scalar_prefetch=3,
            grid=(1,),
            in_specs=[
                pl.BlockSpec(memory_space=pltpu.MemorySpace.HBM),
                pl.BlockSpec(memory_space=pltpu.MemorySpace.HBM),
            ],
            out_specs=[
                pl.BlockSpec((TC_ROWS,), lambda i, *_: (0,)),
                pl.BlockSpec((TC_ROWS,), lambda i, *_: (0,)),
            ],
            scratch_shapes=[
                pltpu.VMEM((TC_ROWS, EMBED_DIM), jnp.float32),
                pltpu.VMEM((TC_ROWS, EMBED_DIM), jnp.float32),
                pltpu.VMEM((TC_ROWS, EMBED_DIM), jnp.float32),
                pltpu.SemaphoreType.DMA,
            ],
        ),
        out_shape=(
            jax.ShapeDtypeStruct((TC_ROWS,), jnp.float32),
            jax.ShapeDtypeStruct((TC_ROWS,), jnp.float32),
        ),
    )
    pos_tc, neg_tc = tc_f(user[SC_ROWS:], pos_item[SC_ROWS:],
                          neg_item[SC_ROWS:], user_emb, item_emb)

    pos = jnp.concatenate([pos_sc, pos_tc])
    neg = jnp.concatenate([neg_sc, neg_tc])
    return (pos, neg)


# speedup vs baseline: 1.1302x; 1.1302x over previous
import functools

import jax
import jax.numpy as jnp
from jax import lax
from jax.experimental import pallas as pl
from jax.experimental.pallas import tpu as pltpu
from jax.experimental.pallas import tpu_sc as plsc

BATCH = 16384
EMBED_DIM = 64
NUM_WORKERS = 32
BPW = BATCH // NUM_WORKERS
NUM_CORES = 2
CHUNK = 256
NCHUNK = BPW // CHUNK


def _bprmf_body(user_hbm, pos_hbm, neg_hbm, uemb_hbm, iemb_hbm,
                pos_out, neg_out,
                uq_v, iq_v, jq_v,
                u_rows, i_rows, j_rows,
                pos_v, neg_v, semu, semi, semj):
    wid = lax.axis_index("s") * NUM_CORES + lax.axis_index("c")
    base = wid * BPW

    pltpu.sync_copy(user_hbm.at[pl.ds(base, BPW)], uq_v)
    pltpu.sync_copy(pos_hbm.at[pl.ds(base, BPW)], iq_v)
    pltpu.sync_copy(neg_hbm.at[pl.ds(base, BPW)], jq_v)

    lanes = lax.iota(jnp.int32, 16)

    for c in range(NCHUNK):
        co = c * CHUNK

        def fire(g, carry, co=co):
            b0 = g * 16
            ru = uq_v[pl.ds(co + b0, 16)]
            ri = iq_v[pl.ds(co + b0, 16)]
            rj = jq_v[pl.ds(co + b0, 16)]
            for b in range(16):
                pltpu.async_copy(uemb_hbm.at[pl.ds(ru[b], 1)],
                                 u_rows.at[pl.ds(b0 + b, 1)], semu)
                pltpu.async_copy(iemb_hbm.at[pl.ds(ri[b], 1)],
                                 i_rows.at[pl.ds(b0 + b, 1)], semi)
                pltpu.async_copy(iemb_hbm.at[pl.ds(rj[b], 1)],
                                 j_rows.at[pl.ds(b0 + b, 1)], semj)
            return carry

        lax.fori_loop(0, CHUNK // 16, fire, 0)

        pltpu.make_async_copy(uemb_hbm.at[pl.ds(0, CHUNK)], u_rows,
                              semu).wait()
        pltpu.make_async_copy(uemb_hbm.at[pl.ds(0, CHUNK)], i_rows,
                              semi).wait()
        pltpu.make_async_copy(uemb_hbm.at[pl.ds(0, CHUNK)], j_rows,
                              semj).wait()

        def group(g, carry, co=co):
            b0 = g * 16
            p_acc = jnp.zeros((16,), jnp.float32)
            n_acc = jnp.zeros((16,), jnp.float32)
            for b in range(16):
                u0 = u_rows[b0 + b, pl.ds(0, 16)]
                u1 = u_rows[b0 + b, pl.ds(16, 16)]
                u2 = u_rows[b0 + b, pl.ds(32, 16)]
                u3 = u_rows[b0 + b, pl.ds(48, 16)]
                i0 = i_rows[b0 + b, pl.ds(0, 16)]
                i1 = i_rows[b0 + b, pl.ds(16, 16)]
                i2 = i_rows[b0 + b, pl.ds(32, 16)]
                i3 = i_rows[b0 + b, pl.ds(48, 16)]
                j0 = j_rows[b0 + b, pl.ds(0, 16)]
                j1 = j_rows[b0 + b, pl.ds(16, 16)]
                j2 = j_rows[b0 + b, pl.ds(32, 16)]
                j3 = j_rows[b0 + b, pl.ds(48, 16)]
                p = (u0 * i0 + u1 * i1) + (u2 * i2 + u3 * i3)
                n = (u0 * j0 + u1 * j1) + (u2 * j2 + u3 * j3)
                sel = lanes == b
                p_acc = jnp.where(sel, jnp.sum(p), p_acc)
                n_acc = jnp.where(sel, jnp.sum(n), n_acc)
            pos_v[pl.ds(co + b0, 16)] = p_acc
            neg_v[pl.ds(co + b0, 16)] = n_acc
            return carry

        lax.fori_loop(0, CHUNK // 16, group, 0)

    pltpu.sync_copy(pos_v, pos_out.at[pl.ds(base, BPW)])
    pltpu.sync_copy(neg_v, neg_out.at[pl.ds(base, BPW)])


@jax.jit
def kernel(user, pos_item, neg_item, user_emb, item_emb):
    mesh = plsc.VectorSubcoreMesh(core_axis_name="c", subcore_axis_name="s")
    f = pl.kernel(
        _bprmf_body,
        mesh=mesh,
        compiler_params=pltpu.CompilerParams(needs_layout_passes=False),
        out_type=(
            jax.ShapeDtypeStruct((BATCH,), jnp.float32),
            jax.ShapeDtypeStruct((BATCH,), jnp.float32),
        ),
        scratch_types=[
            pltpu.VMEM((BPW,), jnp.int32),
            pltpu.VMEM((BPW,), jnp.int32),
            pltpu.VMEM((BPW,), jnp.int32),
            pltpu.VMEM((CHUNK, EMBED_DIM), jnp.float32),
            pltpu.VMEM((CHUNK, EMBED_DIM), jnp.float32),
            pltpu.VMEM((CHUNK, EMBED_DIM), jnp.float32),
            pltpu.VMEM((BPW,), jnp.float32),
            pltpu.VMEM((BPW,), jnp.float32),
            pltpu.SemaphoreType.DMA,
            pltpu.SemaphoreType.DMA,
            pltpu.SemaphoreType.DMA,
        ],
    )
    return f(user, pos_item, neg_item, user_emb, item_emb)
